# baseline (speedup 1.0000x reference)
import jax
import jax.numpy as jnp
from jax import lax
from jax.experimental import pallas as pl
from jax.experimental.pallas import tpu as pltpu
from jax.experimental.pallas import tpu_sc as plsc

_D = 64
_SCALE = 8.0
_NW = 32
_CH = 128
_NBUF = 8
_ROW_UNROLL = 8

_V = 1000000
_SLAB = 256
_NSLAB = _V // _SLAB
_SLAB_EXTRA = _NSLAB % _NW
_NSLAB_EVEN = _NSLAB - _SLAB_EXTRA
_TAIL = _V - _NSLAB * _SLAB

_mesh = plsc.VectorSubcoreMesh(core_axis_name="core",
                               subcore_axis_name="subcore")


def _transpose_table(table_t, tail_lin):
    @pl.kernel(
        out_type=jax.ShapeDtypeStruct((_V * _D,), jnp.float32),
        mesh=_mesh,
        compiler_params=pltpu.CompilerParams(use_tc_tiling_on_sc=True,
                                             needs_layout_passes=False),
        scratch_types=(
            [pltpu.VMEM((_D, 2, 128), jnp.float32) for _ in range(2)]
            + [pltpu.VMEM((_SLAB * _D,), jnp.float32) for _ in range(2)]
            + [pltpu.SemaphoreType.DMA for _ in range(4)]
        ),
    )
    def kern(tbl_hbm, tail_hbm, out_hbm, slab0, slab1, pair0, pair1,
             isem0, isem1, osem0, osem1):
        slabs = (slab0, slab1)
        pairs = (pair0, pair1)
        isems = (isem0, isem1)
        osems = (osem0, osem1)

        wid = lax.axis_index("subcore") * 2 + lax.axis_index("core")
        n_my = _NSLAB_EVEN // _NW

        def col0_of(k):
            return (wid + k * _NW) * _SLAB

        def issue_in(k, buf, sem):
            c0 = col0_of(k)
            pltpu.async_copy(tbl_hbm.at[:, pl.ds(c0, 128)],
                             buf.at[:, 0, :], sem)
            pltpu.async_copy(tbl_hbm.at[:, pl.ds(c0 + 128, 128)],
                             buf.at[:, 1, :], sem)

        def wait_in(buf, sem):
            pltpu.make_async_copy(tbl_hbm.at[:, pl.ds(0, 128)],
                                  buf.at[:, 0, :], sem).wait()
            pltpu.make_async_copy(tbl_hbm.at[:, pl.ds(0, 128)],
                                  buf.at[:, 1, :], sem).wait()

        iota = lax.iota(jnp.int32, 16)

        def transpose(slab, pair):
            for ib in range(16):
                t, j0 = divmod(ib * 16, 128)
                off = (iota + (ib * 16)) * _D

                @pl.loop(0, _D, step=4)
                def _(c):
                    for dc in range(4):
                        v = slab[c + dc, t, pl.ds(j0, 16)]
                        plsc.store_scatter(pair, [off + (c + dc)], v)

        def out_words(k):
            return out_hbm.at[pl.ds(col0_of(k) * _D, _SLAB * _D)]

        issue_in(0, slabs[0], isems[0])

        @pl.loop(0, n_my, step=2)
        def _(s):
            for p in range(2):
                k = s + p
                wait_in(slabs[p], isems[p])

                @pl.when(k + 1 < n_my)
                def _():
                    issue_in(k + 1, slabs[1 - p], isems[1 - p])

                @pl.when(k >= 2)
                def _():
                    pltpu.make_async_copy(pairs[p], out_words(k - 2),
                                          osems[p]).wait()

                transpose(slabs[p], pairs[p])
                pltpu.async_copy(pairs[p], out_words(k), osems[p])

        @pl.when(n_my >= 2)
        def _():
            pltpu.make_async_copy(pairs[0], out_words(n_my - 2),
                                  osems[0]).wait()
            pltpu.make_async_copy(pairs[1], out_words(n_my - 1),
                                  osems[1]).wait()

        @pl.when(wid < _SLAB_EXTRA)
        def _():
            c0 = (_NSLAB_EVEN + wid) * _SLAB
            pltpu.sync_copy(tbl_hbm.at[:, pl.ds(c0, 128)], slab0.at[:, 0, :])
            pltpu.sync_copy(tbl_hbm.at[:, pl.ds(c0 + 128, 128)],
                            slab0.at[:, 1, :])
            transpose(slab0, pair0)
            pltpu.sync_copy(pair0,
                            out_hbm.at[pl.ds(c0 * _D, _SLAB * _D)])

        @pl.when(wid == _SLAB_EXTRA)
        def _():
            pltpu.sync_copy(tail_hbm,
                            out_hbm.at[pl.ds(_NSLAB * _SLAB * _D,
                                             _TAIL * _D)])

    return kern(table_t, tail_lin)


def _gather_scaled(table_lin, idx_flat, n_idx):
    n_per_w = n_idx // _NW
    n_chunk = n_per_w // _CH

    @pl.kernel(
        out_type=jax.ShapeDtypeStruct((n_idx, 2 * _D), jnp.float32),
        mesh=_mesh,
        compiler_params=pltpu.CompilerParams(use_tc_tiling_on_sc=False),
        scratch_types=(
            [pltpu.VMEM((n_per_w,), jnp.int32)]
            + [pltpu.VMEM((_CH, _D), jnp.float32) for _ in range(_NBUF)]
            + [pltpu.SemaphoreType.DMA for _ in range(2 * _NBUF)]
        ),
    )
    def kern(table_hbm, idx_hbm, out_hbm, idx_v, *rest):
        bufs = rest[:_NBUF]
        gsems = rest[_NBUF:2 * _NBUF]
        osems = rest[2 * _NBUF:]

        wid = lax.axis_index("subcore") * 2 + lax.axis_index("core")
        base = wid * n_per_w

        pltpu.sync_copy(idx_hbm.at[pl.ds(base, n_per_w)], idx_v)

        def issue_gather(ch, buf, sem):
            for j in range(_CH // 16):
                iv = idx_v[pl.ds(ch * _CH + j * 16, 16)]
                pltpu.async_copy(table_hbm.at[iv],
                                 buf.at[pl.ds(j * 16, 16), :], sem)

        def drain_gather(buf, sem):
            for j in range(_CH // 16):
                iv = idx_v[pl.ds(j * 16, 16)]
                pltpu.make_async_copy(table_hbm.at[iv],
                                      buf.at[pl.ds(j * 16, 16), :],
                                      sem).wait()

        def scale(buf):
            @pl.loop(0, _CH, step=_ROW_UNROLL)
            def _(r):
                for dr in range(_ROW_UNROLL):
                    for c in range(_D // 16):
                        slc = (pl.ds(r + dr, 1), pl.ds(c * 16, 16))
                        buf.at[*slc][...] = buf.at[*slc][...] * _SCALE

        def out_rows(ch):
            return out_hbm.at[pl.ds(base + ch * _CH, _CH), pl.ds(0, _D)]

        for p in range(_NBUF - 1):
            issue_gather(p, bufs[p], gsems[p])

        @pl.loop(0, n_chunk, step=_NBUF)
        def _(c):
            for p in range(_NBUF):
                ch = c + p
                drain_gather(bufs[p], gsems[p])
                scale(bufs[p])
                pltpu.async_copy(bufs[p], out_rows(ch), osems[p])

                q = (p + _NBUF - 1) % _NBUF
                nxt = ch + _NBUF - 1

                @pl.when(nxt < n_chunk)
                def _():
                    @pl.when(nxt >= _NBUF)
                    def _():
                        pltpu.make_async_copy(bufs[q], out_rows(nxt),
                                              osems[q]).wait()

                    issue_gather(nxt, bufs[q], gsems[q])

        for p in range(_NBUF):
            pltpu.make_async_copy(bufs[p], out_rows(p), osems[p]).wait()

    return kern(table_lin, idx_flat)


@jax.jit
def _run(x, table):
    b, s = x.shape
    n_idx = b * s
    tail_lin = table[_NSLAB * _SLAB:, :].reshape(-1)
    table_lin = _transpose_table(jnp.swapaxes(table, 0, 1), tail_lin)
    out_pad = _gather_scaled(table_lin.reshape(_V, _D), x.reshape(-1),
                             n_idx)
    return out_pad[:, :_D].reshape(b, s, _D)


def kernel(x, table):
    return _run(x, table)


# speedup vs baseline: 1.2566x; 1.2566x over previous
import jax
import jax.numpy as jnp
from jax import lax
from jax.experimental import pallas as pl
from jax.experimental.pallas import tpu as pltpu
from jax.experimental.pallas import tpu_sc as plsc

_D = 64
_SCALE = 8.0
_NW = 32
_CH = 128
_NBUF = 8
_ROW_UNROLL = 8

_V = 1000000
_SLAB = 256
_NSLAB = _V // _SLAB
_SLAB_EXTRA = _NSLAB % _NW
_NSLAB_EVEN = _NSLAB - _SLAB_EXTRA
_TAIL = _V - _NSLAB * _SLAB

_mesh = plsc.VectorSubcoreMesh(core_axis_name="core",
                               subcore_axis_name="subcore")


def _transpose_table(table_t, tail_lin):
    @pl.kernel(
        out_type=jax.ShapeDtypeStruct((_V * _D,), jnp.float32),
        mesh=_mesh,
        compiler_params=pltpu.CompilerParams(use_tc_tiling_on_sc=True,
                                             needs_layout_passes=False),
        scratch_types=(
            [pltpu.VMEM((_D, 2, 128), jnp.float32) for _ in range(2)]
            + [pltpu.VMEM((_SLAB * _D,), jnp.float32) for _ in range(2)]
            + [pltpu.SemaphoreType.DMA for _ in range(4)]
        ),
    )
    def kern(tbl_hbm, tail_hbm, out_hbm, slab0, slab1, pair0, pair1,
             isem0, isem1, osem0, osem1):
        slabs = (slab0, slab1)
        pairs = (pair0, pair1)
        isems = (isem0, isem1)
        osems = (osem0, osem1)

        wid = lax.axis_index("subcore") * 2 + lax.axis_index("core")
        n_my = _NSLAB_EVEN // _NW

        def col0_of(k):
            return (wid + k * _NW) * _SLAB

        def issue_in(k, buf, sem):
            c0 = col0_of(k)
            pltpu.async_copy(tbl_hbm.at[:, pl.ds(c0, 128)],
                             buf.at[:, 0, :], sem)
            pltpu.async_copy(tbl_hbm.at[:, pl.ds(c0 + 128, 128)],
                             buf.at[:, 1, :], sem)

        def wait_in(buf, sem):
            pltpu.make_async_copy(tbl_hbm.at[:, pl.ds(0, 128)],
                                  buf.at[:, 0, :], sem).wait()
            pltpu.make_async_copy(tbl_hbm.at[:, pl.ds(0, 128)],
                                  buf.at[:, 1, :], sem).wait()

        iota = lax.iota(jnp.int32, 16)

        def transpose(slab, pair):
            for ib in range(16):
                t, j0 = divmod(ib * 16, 128)
                off = (iota + (ib * 16)) * _D

                @plsc.parallel_loop(0, _D, step=4, unroll=2)
                def _(c):
                    for dc in range(4):
                        v = slab[c + dc, t, pl.ds(j0, 16)]
                        plsc.store_scatter(pair, [off + (c + dc)], v)

        def out_words(k):
            return out_hbm.at[pl.ds(col0_of(k) * _D, _SLAB * _D)]

        issue_in(0, slabs[0], isems[0])

        @pl.loop(0, n_my, step=2)
        def _(s):
            for p in range(2):
                k = s + p
                wait_in(slabs[p], isems[p])

                @pl.when(k + 1 < n_my)
                def _():
                    issue_in(k + 1, slabs[1 - p], isems[1 - p])

                @pl.when(k >= 2)
                def _():
                    pltpu.make_async_copy(pairs[p], out_words(k - 2),
                                          osems[p]).wait()

                transpose(slabs[p], pairs[p])
                pltpu.async_copy(pairs[p], out_words(k), osems[p])

        @pl.when(n_my >= 2)
        def _():
            pltpu.make_async_copy(pairs[0], out_words(n_my - 2),
                                  osems[0]).wait()
            pltpu.make_async_copy(pairs[1], out_words(n_my - 1),
                                  osems[1]).wait()

        @pl.when(wid < _SLAB_EXTRA)
        def _():
            c0 = (_NSLAB_EVEN + wid) * _SLAB
            pltpu.sync_copy(tbl_hbm.at[:, pl.ds(c0, 128)], slab0.at[:, 0, :])
            pltpu.sync_copy(tbl_hbm.at[:, pl.ds(c0 + 128, 128)],
                            slab0.at[:, 1, :])
            transpose(slab0, pair0)
            pltpu.sync_copy(pair0,
                            out_hbm.at[pl.ds(c0 * _D, _SLAB * _D)])

        @pl.when(wid == _SLAB_EXTRA)
        def _():
            pltpu.sync_copy(tail_hbm,
                            out_hbm.at[pl.ds(_NSLAB * _SLAB * _D,
                                             _TAIL * _D)])

    return kern(table_t, tail_lin)


def _gather_scaled(table_lin, idx_flat, n_idx):
    n_per_w = n_idx // _NW
    n_chunk = n_per_w // _CH

    @pl.kernel(
        out_type=jax.ShapeDtypeStruct((n_idx, 2 * _D), jnp.float32),
        mesh=_mesh,
        compiler_params=pltpu.CompilerParams(use_tc_tiling_on_sc=False),
        scratch_types=(
            [pltpu.VMEM((n_per_w,), jnp.int32)]
            + [pltpu.VMEM((_CH, _D), jnp.float32) for _ in range(_NBUF)]
            + [pltpu.SemaphoreType.DMA for _ in range(2 * _NBUF)]
        ),
    )
    def kern(table_hbm, idx_hbm, out_hbm, idx_v, *rest):
        bufs = rest[:_NBUF]
        gsems = rest[_NBUF:2 * _NBUF]
        osems = rest[2 * _NBUF:]

        wid = lax.axis_index("subcore") * 2 + lax.axis_index("core")
        base = wid * n_per_w

        pltpu.sync_copy(idx_hbm.at[pl.ds(base, n_per_w)], idx_v)

        def issue_gather(ch, buf, sem):
            for j in range(_CH // 16):
                iv = idx_v[pl.ds(ch * _CH + j * 16, 16)]
                pltpu.async_copy(table_hbm.at[iv],
                                 buf.at[pl.ds(j * 16, 16), :], sem)

        def drain_gather(buf, sem):
            for j in range(_CH // 16):
                iv = idx_v[pl.ds(j * 16, 16)]
                pltpu.make_async_copy(table_hbm.at[iv],
                                      buf.at[pl.ds(j * 16, 16), :],
                                      sem).wait()

        def scale(buf):
            @pl.loop(0, _CH, step=_ROW_UNROLL)
            def _(r):
                for dr in range(_ROW_UNROLL):
                    for c in range(_D // 16):
                        slc = (pl.ds(r + dr, 1), pl.ds(c * 16, 16))
                        buf.at[*slc][...] = buf.at[*slc][...] * _SCALE

        def out_rows(ch):
            return out_hbm.at[pl.ds(base + ch * _CH, _CH), pl.ds(0, _D)]

        for p in range(_NBUF - 1):
            issue_gather(p, bufs[p], gsems[p])

        @pl.loop(0, n_chunk, step=_NBUF)
        def _(c):
            for p in range(_NBUF):
                ch = c + p
                drain_gather(bufs[p], gsems[p])
                scale(bufs[p])
                pltpu.async_copy(bufs[p], out_rows(ch), osems[p])

                q = (p + _NBUF - 1) % _NBUF
                nxt = ch + _NBUF - 1

                @pl.when(nxt < n_chunk)
                def _():
                    @pl.when(nxt >= _NBUF)
                    def _():
                        pltpu.make_async_copy(bufs[q], out_rows(nxt),
                                              osems[q]).wait()

                    issue_gather(nxt, bufs[q], gsems[q])

        for p in range(_NBUF):
            pltpu.make_async_copy(bufs[p], out_rows(p), osems[p]).wait()

    return kern(table_lin, idx_flat)


@jax.jit
def _run(x, table):
    b, s = x.shape
    n_idx = b * s
    tail_lin = table[_NSLAB * _SLAB:, :].reshape(-1)
    table_lin = _transpose_table(jnp.swapaxes(table, 0, 1), tail_lin)
    out_pad = _gather_scaled(table_lin.reshape(_V, _D), x.reshape(-1),
                             n_idx)
    return out_pad[:, :_D].reshape(b, s, _D)


def kernel(x, table):
    return _run(x, table)


# speedup vs baseline: 1.4125x; 1.1240x over previous
import jax
import jax.numpy as jnp
from jax import lax
from jax.experimental import pallas as pl
from jax.experimental.pallas import tpu as pltpu
from jax.experimental.pallas import tpu_sc as plsc

_D = 64
_SCALE = 8.0
_NW = 32
_CH = 128
_NBUF = 8
_ROW_UNROLL = 8

_V = 1000000
_SLAB = 256
_NSLAB = _V // _SLAB
_SLAB_EXTRA = _NSLAB % _NW
_NSLAB_EVEN = _NSLAB - _SLAB_EXTRA
_TAIL = _V - _NSLAB * _SLAB

_mesh = plsc.VectorSubcoreMesh(core_axis_name="core",
                               subcore_axis_name="subcore")


def _transpose_table(table_t, tail_lin):
    @pl.kernel(
        out_type=jax.ShapeDtypeStruct((_V * _D,), jnp.float32),
        mesh=_mesh,
        compiler_params=pltpu.CompilerParams(use_tc_tiling_on_sc=True,
                                             needs_layout_passes=False),
        scratch_types=(
            [pltpu.VMEM((_D, 257), jnp.float32) for _ in range(2)]
            + [pltpu.VMEM((_SLAB * _D,), jnp.float32) for _ in range(2)]
            + [pltpu.SemaphoreType.DMA for _ in range(4)]
        ),
    )
    def kern(tbl_hbm, tail_hbm, out_hbm, slab0, slab1, pair0, pair1,
             isem0, isem1, osem0, osem1):
        slabs = (slab0, slab1)
        pairs = (pair0, pair1)
        isems = (isem0, isem1)
        osems = (osem0, osem1)

        wid = lax.axis_index("subcore") * 2 + lax.axis_index("core")
        n_my = _NSLAB_EVEN // _NW

        def col0_of(k):
            return (wid + k * _NW) * _SLAB

        def issue_in(k, buf, sem):
            c0 = col0_of(k)
            pltpu.async_copy(tbl_hbm.at[:, pl.ds(c0, 128)],
                             buf.at[:, pl.ds(0, 128)], sem)
            pltpu.async_copy(tbl_hbm.at[:, pl.ds(c0 + 128, 128)],
                             buf.at[:, pl.ds(128, 128)], sem)

        def wait_in(buf, sem):
            pltpu.make_async_copy(tbl_hbm.at[:, pl.ds(0, 128)],
                                  buf.at[:, pl.ds(0, 128)], sem).wait()
            pltpu.make_async_copy(tbl_hbm.at[:, pl.ds(0, 128)],
                                  buf.at[:, pl.ds(128, 128)], sem).wait()

        iota = lax.iota(jnp.int32, 16)
        c_idx = [iota + (j * 16) for j in range(4)]

        def transpose(slab, pair):
            @plsc.parallel_loop(0, _SLAB, step=2, unroll=2)
            def _(i):
                for di in range(2):
                    ii = i + di
                    i_vec = iota * 0 + ii
                    for j in range(4):
                        v = plsc.load_gather(slab, [c_idx[j], i_vec])
                        pair[pl.ds(ii * _D + j * 16, 16)] = v

        def out_words(k):
            return out_hbm.at[pl.ds(col0_of(k) * _D, _SLAB * _D)]

        issue_in(0, slabs[0], isems[0])

        @pl.loop(0, n_my, step=2)
        def _(s):
            for p in range(2):
                k = s + p
                wait_in(slabs[p], isems[p])

                @pl.when(k + 1 < n_my)
                def _():
                    issue_in(k + 1, slabs[1 - p], isems[1 - p])

                @pl.when(k >= 2)
                def _():
                    pltpu.make_async_copy(pairs[p], out_words(k - 2),
                                          osems[p]).wait()

                transpose(slabs[p], pairs[p])
                pltpu.async_copy(pairs[p], out_words(k), osems[p])

        @pl.when(n_my >= 2)
        def _():
            pltpu.make_async_copy(pairs[0], out_words(n_my - 2),
                                  osems[0]).wait()
            pltpu.make_async_copy(pairs[1], out_words(n_my - 1),
                                  osems[1]).wait()

        @pl.when(wid < _SLAB_EXTRA)
        def _():
            c0 = (_NSLAB_EVEN + wid) * _SLAB
            pltpu.sync_copy(tbl_hbm.at[:, pl.ds(c0, 128)],
                            slab0.at[:, pl.ds(0, 128)])
            pltpu.sync_copy(tbl_hbm.at[:, pl.ds(c0 + 128, 128)],
                            slab0.at[:, pl.ds(128, 128)])
            transpose(slab0, pair0)
            pltpu.sync_copy(pair0,
                            out_hbm.at[pl.ds(c0 * _D, _SLAB * _D)])

        @pl.when(wid == _SLAB_EXTRA)
        def _():
            pltpu.sync_copy(tail_hbm,
                            out_hbm.at[pl.ds(_NSLAB * _SLAB * _D,
                                             _TAIL * _D)])

    return kern(table_t, tail_lin)


def _gather_scaled(table_lin, idx_flat, n_idx):
    n_per_w = n_idx // _NW
    n_chunk = n_per_w // _CH

    @pl.kernel(
        out_type=jax.ShapeDtypeStruct((n_idx, 2 * _D), jnp.float32),
        mesh=_mesh,
        compiler_params=pltpu.CompilerParams(use_tc_tiling_on_sc=False),
        scratch_types=(
            [pltpu.VMEM((n_per_w,), jnp.int32)]
            + [pltpu.VMEM((_CH, _D), jnp.float32) for _ in range(_NBUF)]
            + [pltpu.SemaphoreType.DMA for _ in range(2 * _NBUF)]
        ),
    )
    def kern(table_hbm, idx_hbm, out_hbm, idx_v, *rest):
        bufs = rest[:_NBUF]
        gsems = rest[_NBUF:2 * _NBUF]
        osems = rest[2 * _NBUF:]

        wid = lax.axis_index("subcore") * 2 + lax.axis_index("core")
        base = wid * n_per_w

        pltpu.sync_copy(idx_hbm.at[pl.ds(base, n_per_w)], idx_v)

        def issue_gather(ch, buf, sem):
            for j in range(_CH // 16):
                iv = idx_v[pl.ds(ch * _CH + j * 16, 16)]
                pltpu.async_copy(table_hbm.at[iv],
                                 buf.at[pl.ds(j * 16, 16), :], sem)

        def drain_gather(buf, sem):
            for j in range(_CH // 16):
                iv = idx_v[pl.ds(j * 16, 16)]
                pltpu.make_async_copy(table_hbm.at[iv],
                                      buf.at[pl.ds(j * 16, 16), :],
                                      sem).wait()

        def scale(buf):
            @pl.loop(0, _CH, step=_ROW_UNROLL)
            def _(r):
                for dr in range(_ROW_UNROLL):
                    for c in range(_D // 16):
                        slc = (pl.ds(r + dr, 1), pl.ds(c * 16, 16))
                        buf.at[*slc][...] = buf.at[*slc][...] * _SCALE

        def out_rows(ch):
            return out_hbm.at[pl.ds(base + ch * _CH, _CH), pl.ds(0, _D)]

        for p in range(_NBUF - 1):
            issue_gather(p, bufs[p], gsems[p])

        @pl.loop(0, n_chunk, step=_NBUF)
        def _(c):
            for p in range(_NBUF):
                ch = c + p
                drain_gather(bufs[p], gsems[p])
                scale(bufs[p])
                pltpu.async_copy(bufs[p], out_rows(ch), osems[p])

                q = (p + _NBUF - 1) % _NBUF
                nxt = ch + _NBUF - 1

                @pl.when(nxt < n_chunk)
                def _():
                    @pl.when(nxt >= _NBUF)
                    def _():
                        pltpu.make_async_copy(bufs[q], out_rows(nxt),
                                              osems[q]).wait()

                    issue_gather(nxt, bufs[q], gsems[q])

        for p in range(_NBUF):
            pltpu.make_async_copy(bufs[p], out_rows(p), osems[p]).wait()

    return kern(table_lin, idx_flat)


@jax.jit
def _run(x, table):
    b, s = x.shape
    n_idx = b * s
    tail_lin = table[_NSLAB * _SLAB:, :].reshape(-1)
    table_lin = _transpose_table(jnp.swapaxes(table, 0, 1), tail_lin)
    out_pad = _gather_scaled(table_lin.reshape(_V, _D), x.reshape(-1),
                             n_idx)
    return out_pad[:, :_D].reshape(b, s, _D)


def kernel(x, table):
    return _run(x, table)


# speedup vs baseline: 1.7215x; 1.2188x over previous
import jax
import jax.numpy as jnp
from jax import lax
from jax.experimental import pallas as pl
from jax.experimental.pallas import tpu as pltpu
from jax.experimental.pallas import tpu_sc as plsc

_D = 64
_SCALE = 8.0
_NW = 32
_SUBW = 128
_CH = 128
_NSTREAM = _CH // _SUBW
_NBUF = 8
_ROW_UNROLL = 8


def _emb_kernel(n_idx: int):
    n_per_w = n_idx // _NW
    n_chunk = n_per_w // _CH
    assert n_per_w % _CH == 0 and n_chunk % _NBUF == 0

    mesh = plsc.VectorSubcoreMesh(core_axis_name="core",
                                  subcore_axis_name="subcore")

    @jax.jit
    def run(idx_flat, table):
        @pl.kernel(
            out_type=jax.ShapeDtypeStruct((n_idx, 2 * _D), jnp.float32),
            mesh=mesh,
            compiler_params=pltpu.CompilerParams(use_tc_tiling_on_sc=False),
            scratch_types=(
                [pltpu.VMEM((n_per_w,), jnp.int32)]
                + [pltpu.VMEM((_CH, _D), jnp.float32) for _ in range(_NBUF)]
                + [pltpu.SemaphoreType.DMA for _ in range(2 * _NBUF)]
            ),
        )
        def kern(table_hbm, idx_hbm, out_hbm, idx_v, *rest):
            bufs = rest[:_NBUF]
            gsems = rest[_NBUF:2 * _NBUF]
            osems = rest[2 * _NBUF:]

            wid = lax.axis_index("subcore") * 2 + lax.axis_index("core")
            base = wid * n_per_w

            pltpu.sync_copy(idx_hbm.at[pl.ds(base, n_per_w)], idx_v)

            def issue_gather(ch, buf, sem):
                for j in range(_CH // 16):
                    iv = idx_v[pl.ds(ch * _CH + j * 16, 16)]
                    pltpu.async_copy(
                        table_hbm.at[iv],
                        buf.at[pl.ds(j * 16, 16), :],
                        sem,
                    )

            def drain_gather(buf, sem):
                for j in range(_CH // 16):
                    iv = idx_v[pl.ds(j * 16, 16)]
                    pltpu.make_async_copy(
                        table_hbm.at[iv],
                        buf.at[pl.ds(j * 16, 16), :],
                        sem,
                    ).wait()

            def scale(buf):
                @pl.loop(0, _CH, step=_ROW_UNROLL)
                def _(r):
                    for dr in range(_ROW_UNROLL):
                        for c in range(_D // 16):
                            slc = (pl.ds(r + dr, 1), pl.ds(c * 16, 16))
                            buf.at[*slc][...] = buf.at[*slc][...] * _SCALE

            def out_rows(ch):
                return out_hbm.at[pl.ds(base + ch * _CH, _CH), pl.ds(0, _D)]

            for p in range(_NBUF - 1):
                issue_gather(p, bufs[p], gsems[p])

            @pl.loop(0, n_chunk, step=_NBUF)
            def _(c):
                for p in range(_NBUF):
                    ch = c + p
                    drain_gather(bufs[p], gsems[p])
                    scale(bufs[p])
                    pltpu.async_copy(bufs[p], out_rows(ch), osems[p])

                    q = (p + _NBUF - 1) % _NBUF
                    nxt = ch + _NBUF - 1

                    @pl.when(nxt < n_chunk)
                    def _():
                        @pl.when(nxt >= _NBUF)
                        def _():
                            pltpu.make_async_copy(
                                bufs[q], out_rows(nxt), osems[q]
                            ).wait()

                        issue_gather(nxt, bufs[q], gsems[q])

            for p in range(_NBUF):
                pltpu.make_async_copy(
                    bufs[p], out_rows(p), osems[p]
                ).wait()

        return kern(table, idx_flat)

    return run


def kernel(x, table):
    b, s = x.shape
    out = _emb_kernel(b * s)(x.reshape(-1), table)
    return out[:, :_D].reshape(b, s, _D)
